# baseline (speedup 1.0000x reference)
import functools

import jax
import jax.numpy as jnp
from jax import lax
from jax.experimental import pallas as pl
from jax.experimental.pallas import tpu as pltpu
from jax.experimental.pallas import tpu_sc as plsc

D_MODEL = 128
PACKED_W = D_MODEL // 2
NUM_CORES = 2
NUM_SUBCORES = 16
NUM_WORKERS = NUM_CORES * NUM_SUBCORES

GATHER_ROWS = 128
GATHERS_PER_CHUNK = 2
CHUNK = GATHER_ROWS * GATHERS_PER_CHUNK
NBUF = 3


def _gather_body(n_chunks, table_hbm, idx_hbm, out_hbm, idx_v, rows_v, gsem, wsem):
    wid = lax.axis_index("s") * NUM_CORES + lax.axis_index("c")
    idx_rows = n_chunks * GATHERS_PER_CHUNK
    pltpu.sync_copy(idx_hbm.at[pl.ds(wid * idx_rows, idx_rows)], idx_v)
    base_row = wid * n_chunks * CHUNK

    def gather_copies(i, b):
        return [
            pltpu.make_async_copy(
                table_hbm.at[idx_v.at[i * GATHERS_PER_CHUNK + k]],
                rows_v.at[b, pl.ds(k * GATHER_ROWS, GATHER_ROWS)],
                gsem.at[b],
            )
            for k in range(GATHERS_PER_CHUNK)
        ]

    def write_copy(i, b):
        return pltpu.make_async_copy(
            rows_v.at[b],
            out_hbm.at[pl.ds(base_row + i * CHUNK, CHUNK)],
            wsem.at[b],
        )

    def fire_gather(i, b):
        for c in gather_copies(i, b):
            c.start()

    def wait_gather(i, b):
        for c in gather_copies(i, b):
            c.wait()

    def step(i, b):
        wait_gather(i, b)
        write_copy(i, b).start()

        @pl.when(i >= 1)
        def _():
            write_copy(i - 1, (b + NBUF - 1) % NBUF).wait()

        @pl.when(i + NBUF - 1 < n_chunks)
        def _():
            fire_gather(i + NBUF - 1, (b + NBUF - 1) % NBUF)

    for j in range(NBUF - 1):
        fire_gather(j, j)

    @pl.loop(0, n_chunks - 1, step=NBUF)
    def _chunk_group(g):
        for b in range(NBUF):
            step(g + b, b)

    step(n_chunks - 1, (n_chunks - 1) % NBUF)
    write_copy(n_chunks - 1, (n_chunks - 1) % NBUF).wait()


@functools.partial(jax.jit, static_argnames=("n_rows", "width"))
def _gather(table, idx2d, n_rows, width):
    n_chunks = n_rows // (NUM_WORKERS * CHUNK)
    mesh = plsc.VectorSubcoreMesh(core_axis_name="c", subcore_axis_name="s")
    run = pl.kernel(
        functools.partial(_gather_body, n_chunks),
        out_type=jax.ShapeDtypeStruct((n_rows, width), jnp.int32),
        mesh=mesh,
        scratch_types=[
            pltpu.VMEM((n_chunks * GATHERS_PER_CHUNK, GATHER_ROWS), jnp.int32),
            pltpu.VMEM((NBUF, CHUNK, width), jnp.int32),
            pltpu.SemaphoreType.DMA((NBUF,)),
            pltpu.SemaphoreType.DMA((NBUF,)),
        ],
        compiler_params=pltpu.CompilerParams(use_tc_tiling_on_sc=False),
    )
    return run(table, idx2d)


def kernel(variate_ids, variate_embed):
    batch, hist = variate_ids.shape
    n_rows = batch * hist
    tab16 = variate_embed.astype(jnp.bfloat16)
    tab_pk = lax.bitcast_convert_type(
        tab16.reshape(-1, PACKED_W, 2), jnp.int32
    )
    idx2d = variate_ids.reshape(n_rows // GATHER_ROWS, GATHER_ROWS)
    idx2d = idx2d.astype(jnp.int32)
    out_pk = _gather(tab_pk, idx2d, n_rows, PACKED_W)
    out16 = lax.bitcast_convert_type(out_pk, jnp.bfloat16)
    return out16.reshape(batch, hist, D_MODEL).astype(jnp.float32)


# speedup vs baseline: 11.0224x; 11.0224x over previous
import functools

import jax
import jax.numpy as jnp
from jax import lax
from jax.experimental import pallas as pl
from jax.experimental.pallas import tpu as pltpu
from jax.experimental.pallas import tpu_sc as plsc

D_MODEL = 128
NUM_CORES = 2
NUM_SUBCORES = 16
NUM_WORKERS = NUM_CORES * NUM_SUBCORES

GATHER_ROWS = 128
GATHERS_PER_CHUNK = 2
CHUNK = GATHER_ROWS * GATHERS_PER_CHUNK
NBUF = 3


def _gather_body(n_chunks, table_hbm, idx_hbm, out_hbm, idx_v, rows_v, gsem, wsem):
    wid = lax.axis_index("s") * NUM_CORES + lax.axis_index("c")
    idx_rows = n_chunks * GATHERS_PER_CHUNK
    pltpu.sync_copy(idx_hbm.at[pl.ds(wid * idx_rows, idx_rows)], idx_v)
    base_row = wid * n_chunks * CHUNK

    def gather_copies(i, b):
        return [
            pltpu.make_async_copy(
                table_hbm.at[idx_v.at[i * GATHERS_PER_CHUNK + k]],
                rows_v.at[b, pl.ds(k * GATHER_ROWS, GATHER_ROWS)],
                gsem.at[b],
            )
            for k in range(GATHERS_PER_CHUNK)
        ]

    def write_copy(i, b):
        return pltpu.make_async_copy(
            rows_v.at[b],
            out_hbm.at[pl.ds(base_row + i * CHUNK, CHUNK)],
            wsem.at[b],
        )

    def fire_gather(i, b):
        for c in gather_copies(i, b):
            c.start()

    def wait_gather(i, b):
        for c in gather_copies(i, b):
            c.wait()

    def step(i, b):
        wait_gather(i, b)
        write_copy(i, b).start()

        @pl.when(i >= 1)
        def _():
            write_copy(i - 1, (b + NBUF - 1) % NBUF).wait()

        @pl.when(i + NBUF - 1 < n_chunks)
        def _():
            fire_gather(i + NBUF - 1, (b + NBUF - 1) % NBUF)

    for j in range(min(NBUF - 1, n_chunks)):
        fire_gather(j, j)

    n_main = ((n_chunks - 1) // NBUF) * NBUF

    @pl.loop(0, n_main, step=NBUF)
    def _chunk_group(g):
        for b in range(NBUF):
            step(g + b, b)

    for i in range(n_main, n_chunks):
        step(i, i % NBUF)
    write_copy(n_chunks - 1, (n_chunks - 1) % NBUF).wait()


@functools.partial(jax.jit, static_argnames=("n_rows",))
def _gather(table, idx2d, n_rows):
    n_chunks = n_rows // (NUM_WORKERS * CHUNK)
    mesh = plsc.VectorSubcoreMesh(core_axis_name="c", subcore_axis_name="s")
    run = pl.kernel(
        functools.partial(_gather_body, n_chunks),
        out_type=jax.ShapeDtypeStruct((n_rows, D_MODEL), jnp.float32),
        mesh=mesh,
        scratch_types=[
            pltpu.VMEM((n_chunks * GATHERS_PER_CHUNK, GATHER_ROWS), jnp.int32),
            pltpu.VMEM((NBUF, CHUNK, D_MODEL), jnp.float32),
            pltpu.SemaphoreType.DMA((NBUF,)),
            pltpu.SemaphoreType.DMA((NBUF,)),
        ],
    )
    return run(table, idx2d)


def kernel(variate_ids, variate_embed):
    batch, hist = variate_ids.shape
    n_rows = batch * hist
    idx2d = variate_ids.reshape(n_rows // GATHER_ROWS, GATHER_ROWS)
    idx2d = idx2d.astype(jnp.int32)
    out = _gather(variate_embed, idx2d, n_rows)
    return out.reshape(batch, hist, D_MODEL)


# speedup vs baseline: 11.0806x; 1.0053x over previous
import functools

import jax
import jax.numpy as jnp
from jax import lax
from jax.experimental import pallas as pl
from jax.experimental.pallas import tpu as pltpu
from jax.experimental.pallas import tpu_sc as plsc

D_MODEL = 128
NUM_CORES = 2
NUM_SUBCORES = 16
NUM_WORKERS = NUM_CORES * NUM_SUBCORES

GATHER_ROWS = 128
GATHERS_PER_CHUNK = 1
CHUNK = GATHER_ROWS * GATHERS_PER_CHUNK
NBUF = 6


def _gather_body(n_chunks, table_hbm, idx_hbm, out_hbm, idx_v, rows_v, gsem, wsem):
    wid = lax.axis_index("s") * NUM_CORES + lax.axis_index("c")
    idx_rows = n_chunks * GATHERS_PER_CHUNK
    pltpu.sync_copy(idx_hbm.at[pl.ds(wid * idx_rows, idx_rows)], idx_v)
    base_row = wid * n_chunks * CHUNK

    def gather_copies(i, b):
        return [
            pltpu.make_async_copy(
                table_hbm.at[idx_v.at[i * GATHERS_PER_CHUNK + k]],
                rows_v.at[b, pl.ds(k * GATHER_ROWS, GATHER_ROWS)],
                gsem.at[b],
            )
            for k in range(GATHERS_PER_CHUNK)
        ]

    def write_copy(i, b):
        return pltpu.make_async_copy(
            rows_v.at[b],
            out_hbm.at[pl.ds(base_row + i * CHUNK, CHUNK)],
            wsem.at[b],
        )

    def fire_gather(i, b):
        for c in gather_copies(i, b):
            c.start()

    def wait_gather(i, b):
        for c in gather_copies(i, b):
            c.wait()

    def step(i, b):
        wait_gather(i, b)
        write_copy(i, b).start()

        @pl.when(i >= 1)
        def _():
            write_copy(i - 1, (b + NBUF - 1) % NBUF).wait()

        @pl.when(i + NBUF - 1 < n_chunks)
        def _():
            fire_gather(i + NBUF - 1, (b + NBUF - 1) % NBUF)

    for j in range(min(NBUF - 1, n_chunks)):
        fire_gather(j, j)

    n_main = ((n_chunks - 1) // NBUF) * NBUF

    @pl.loop(0, n_main, step=NBUF)
    def _chunk_group(g):
        for b in range(NBUF):
            step(g + b, b)

    for i in range(n_main, n_chunks):
        step(i, i % NBUF)
    write_copy(n_chunks - 1, (n_chunks - 1) % NBUF).wait()


@functools.partial(jax.jit, static_argnames=("n_rows",))
def _gather(table, idx2d, n_rows):
    n_chunks = n_rows // (NUM_WORKERS * CHUNK)
    mesh = plsc.VectorSubcoreMesh(core_axis_name="c", subcore_axis_name="s")
    run = pl.kernel(
        functools.partial(_gather_body, n_chunks),
        out_type=jax.ShapeDtypeStruct((n_rows, D_MODEL), jnp.float32),
        mesh=mesh,
        scratch_types=[
            pltpu.VMEM((n_chunks * GATHERS_PER_CHUNK, GATHER_ROWS), jnp.int32),
            pltpu.VMEM((NBUF, CHUNK, D_MODEL), jnp.float32),
            pltpu.SemaphoreType.DMA((NBUF,)),
            pltpu.SemaphoreType.DMA((NBUF,)),
        ],
    )
    return run(table, idx2d)


def kernel(variate_ids, variate_embed):
    batch, hist = variate_ids.shape
    n_rows = batch * hist
    idx2d = variate_ids.reshape(n_rows // GATHER_ROWS, GATHER_ROWS)
    idx2d = idx2d.astype(jnp.int32)
    out = _gather(variate_embed, idx2d, n_rows)
    return out.reshape(batch, hist, D_MODEL)
